# Initial kernel scaffold; baseline (speedup 1.0000x reference)
import functools

import jax
import jax.numpy as jnp
from jax import lax
from jax.experimental import pallas as pl
from jax.experimental.pallas import tpu as pltpu
from jax.experimental.pallas import tpu_sc as plsc

B, T, D, L, V = 4096, 200, 64, 4, 100000
VE = V + 2
N = B * T
NC, NS = 2, 16
NW = NC * NS
PER_W = N // NW
CH = 128
NCH = PER_W // CH
GRP = 4
NROUND = NCH // GRP
NBUF = 2 * GRP

ROW_BR = 2048


def _aug_body(e0_ref, e1_ref, e2_ref, e3_ref, lv_ref, out_ref):
    lv = lv_ref[...]
    for l, e_ref in enumerate((e0_ref, e1_ref, e2_ref, e3_ref)):
        out_ref[l, :, :] = e_ref[...] + lv[l, :]


def _build_table(emb0, emb1, emb2, emb3, level_embed):
    grid = (pl.cdiv(VE, ROW_BR),)
    aug = pl.pallas_call(
        _aug_body,
        grid=grid,
        in_specs=[pl.BlockSpec((ROW_BR, D), lambda i: (i, 0)) for _ in range(4)]
        + [pl.BlockSpec((L, D), lambda i: (0, 0))],
        out_specs=pl.BlockSpec((L, ROW_BR, D), lambda i: (0, i, 0)),
        out_shape=jax.ShapeDtypeStruct((L, VE, D), jnp.float32),
    )(emb0, emb1, emb2, emb3, level_embed)
    return aug.reshape(L * VE, D)


def _idx_body(lv_ref, tk_ref, out_ref):
    out_ref[...] = lv_ref[...] * VE + tk_ref[...]


def _build_idx(level_ids, token_ids):
    IB = 512
    gidx = pl.pallas_call(
        _idx_body,
        grid=(B // IB,),
        in_specs=[pl.BlockSpec((IB, T), lambda i: (i, 0)) for _ in range(2)],
        out_specs=pl.BlockSpec((IB, T), lambda i: (i, 0)),
        out_shape=jax.ShapeDtypeStruct((B, T), jnp.int32),
    )(level_ids, token_ids)
    return gidx.reshape(N // CH, CH)


_MESH = plsc.VectorSubcoreMesh(
    core_axis_name="c", subcore_axis_name="s", num_cores=NC, num_subcores=NS
)


@functools.partial(
    pl.kernel,
    out_type=jax.ShapeDtypeStruct((N, D), jnp.float32),
    mesh=_MESH,
    scratch_types=(
        [pltpu.VMEM((NCH, CH), jnp.int32)]
        + [pltpu.VMEM((CH, D), jnp.float32) for _ in range(NBUF)]
        + [pltpu.SemaphoreType.DMA for _ in range(4)]
    ),
)
def _sc_gather(idx_hbm, table_hbm, out_hbm, idx_v,
               b0, b1, b2, b3, b4, b5, b6, b7, gs0, gs1, ws0, ws1):
    bufs = ((b0, b1, b2, b3), (b4, b5, b6, b7))
    gsem = (gs0, gs1)
    wsem = (ws0, ws1)
    wid = lax.axis_index("s") * NC + lax.axis_index("c")
    row0 = wid * NCH
    out0 = wid * PER_W

    pltpu.sync_copy(idx_hbm.at[pl.ds(row0, NCH)], idx_v)

    for g in range(2):
        for b in range(GRP):
            pltpu.async_copy(
                table_hbm.at[idx_v.at[g * GRP + b]], bufs[g][b], gsem[g]
            )

    def do_round(r, g):
        j0 = r * GRP
        for b in range(GRP):
            pltpu.make_async_copy(
                table_hbm.at[idx_v.at[0]], bufs[g][b], gsem[g]
            ).wait()
        for b in range(GRP):
            pltpu.async_copy(
                bufs[g][b],
                out_hbm.at[pl.ds(out0 + (j0 + b) * CH, CH)],
                wsem[g],
            )
        for b in range(GRP):
            pltpu.make_async_copy(
                bufs[g][b], out_hbm.at[pl.ds(out0, CH)], wsem[g]
            ).wait()
        nj0 = (r + 2) * GRP

        @pl.when(nj0 < NCH)
        def _():
            for b in range(GRP):
                pltpu.async_copy(
                    table_hbm.at[idx_v.at[nj0 + b]], bufs[g][b], gsem[g]
                )

    def outer(i, carry):
        do_round(2 * i, 0)
        do_round(2 * i + 1, 1)
        return carry

    lax.fori_loop(0, NROUND // 2, outer, None)


def kernel(level_ids, token_ids, emb0, emb1, emb2, emb3, level_embed):
    level_ids = level_ids.astype(jnp.int32)
    token_ids = token_ids.astype(jnp.int32)
    table = _build_table(emb0, emb1, emb2, emb3, level_embed)
    idx2d = _build_idx(level_ids, token_ids)
    out = _sc_gather(idx2d, table)
    return out.reshape(B, T, D)


# speedup vs baseline: 5.7638x; 5.7638x over previous
import functools

import jax
import jax.numpy as jnp
from jax import lax
from jax.experimental import pallas as pl
from jax.experimental.pallas import tpu as pltpu
from jax.experimental.pallas import tpu_sc as plsc

B, T, D, L, V = 4096, 200, 64, 4, 100000
VE = V + 2
N = B * T
NC, NS = 2, 16
NW = NC * NS
PER_W = N // NW
CH = 128
NCH = PER_W // CH
GRP = 4
NROUND = NCH // GRP
NBUF = 2 * GRP

ROW_BR = 2048


def _aug_body(e0_ref, e1_ref, e2_ref, e3_ref, lv_ref, out_ref):
    lv = lv_ref[...]
    for l, e_ref in enumerate((e0_ref, e1_ref, e2_ref, e3_ref)):
        out_ref[l, :, :] = e_ref[...] + lv[l, :]


def _build_table(emb0, emb1, emb2, emb3, level_embed):
    grid = (pl.cdiv(VE, ROW_BR),)
    aug = pl.pallas_call(
        _aug_body,
        grid=grid,
        in_specs=[pl.BlockSpec((ROW_BR, D), lambda i: (i, 0)) for _ in range(4)]
        + [pl.BlockSpec((L, D), lambda i: (0, 0))],
        out_specs=pl.BlockSpec((L, ROW_BR, D), lambda i: (0, i, 0)),
        out_shape=jax.ShapeDtypeStruct((L, VE, D), jnp.float32),
    )(emb0, emb1, emb2, emb3, level_embed)
    return aug.reshape(L * VE, D)


def _idx_body(lv_ref, tk_ref, out_ref):
    out_ref[...] = lv_ref[...] * VE + tk_ref[...]


def _build_idx(level_ids, token_ids):
    IB = 512
    gidx = pl.pallas_call(
        _idx_body,
        grid=(B // IB,),
        in_specs=[pl.BlockSpec((IB, T), lambda i: (i, 0)) for _ in range(2)],
        out_specs=pl.BlockSpec((IB, T), lambda i: (i, 0)),
        out_shape=jax.ShapeDtypeStruct((B, T), jnp.int32),
    )(level_ids, token_ids)
    return gidx.reshape(N // CH, CH)


_MESH = plsc.VectorSubcoreMesh(
    core_axis_name="c", subcore_axis_name="s", num_cores=NC, num_subcores=NS
)


@functools.partial(
    pl.kernel,
    out_type=jax.ShapeDtypeStruct((N, D), jnp.float32),
    mesh=_MESH,
    scratch_types=(
        [pltpu.VMEM((NCH, CH), jnp.int32)]
        + [pltpu.VMEM((CH, D), jnp.float32) for _ in range(NBUF)]
        + [pltpu.SemaphoreType.DMA for _ in range(4)]
    ),
    compiler_params=pltpu.CompilerParams(use_tc_tiling_on_sc=False),
)
def _sc_gather(idx_hbm, table_hbm, out_hbm, idx_v,
               b0, b1, b2, b3, b4, b5, b6, b7, gs0, gs1, ws0, ws1):
    bufs = ((b0, b1, b2, b3), (b4, b5, b6, b7))
    gsem = (gs0, gs1)
    wsem = (ws0, ws1)
    wid = lax.axis_index("s") * NC + lax.axis_index("c")
    row0 = wid * NCH
    out0 = wid * PER_W

    pltpu.sync_copy(idx_hbm.at[pl.ds(row0, NCH)], idx_v)

    for g in range(2):
        for b in range(GRP):
            pltpu.async_copy(
                table_hbm.at[idx_v.at[g * GRP + b]], bufs[g][b], gsem[g]
            )

    def do_round(r, g):
        j0 = r * GRP
        for b in range(GRP):
            pltpu.make_async_copy(
                table_hbm.at[idx_v.at[0]], bufs[g][b], gsem[g]
            ).wait()
        for b in range(GRP):
            pltpu.async_copy(
                bufs[g][b],
                out_hbm.at[pl.ds(out0 + (j0 + b) * CH, CH)],
                wsem[g],
            )
        for b in range(GRP):
            pltpu.make_async_copy(
                bufs[g][b], out_hbm.at[pl.ds(out0, CH)], wsem[g]
            ).wait()
        nj0 = (r + 2) * GRP

        @pl.when(nj0 < NCH)
        def _():
            for b in range(GRP):
                pltpu.async_copy(
                    table_hbm.at[idx_v.at[nj0 + b]], bufs[g][b], gsem[g]
                )

    def outer(i, carry):
        do_round(2 * i, 0)
        do_round(2 * i + 1, 1)
        return carry

    lax.fori_loop(0, NROUND // 2, outer, None)


def kernel(level_ids, token_ids, emb0, emb1, emb2, emb3, level_embed):
    level_ids = level_ids.astype(jnp.int32)
    token_ids = token_ids.astype(jnp.int32)
    table = _build_table(emb0, emb1, emb2, emb3, level_embed)
    idx2d = _build_idx(level_ids, token_ids)
    out = _sc_gather(idx2d, table)
    return out.reshape(B, T, D)


# speedup vs baseline: 12.5389x; 2.1755x over previous
import functools

import jax
import jax.numpy as jnp
from jax import lax
from jax.experimental import pallas as pl
from jax.experimental.pallas import tpu as pltpu
from jax.experimental.pallas import tpu_sc as plsc

B, T, D, L, V = 4096, 200, 64, 4, 100000
VE = V + 2
N = B * T
NC, NS = 2, 16
NW = NC * NS
PER_W = N // NW
CH = 128
NCH = PER_W // CH
GRP = 2
NROUND = NCH // GRP
NBUF = 2 * GRP

BC = 512
VEP = 100352
DP = 2 * D


def _aug_body(e0_ref, e1_ref, e2_ref, e3_ref, lv_ref, out_ref):
    lv = lv_ref[...]
    for l, e_ref in enumerate((e0_ref, e1_ref, e2_ref, e3_ref)):
        row = e_ref[...].T + lv[l, :]
        out_ref[l, :, :] = jnp.concatenate([row, row], axis=-1)


def _build_table(emb0, emb1, emb2, emb3, level_embed):
    grid = (VEP // BC,)
    aug = pl.pallas_call(
        _aug_body,
        grid=grid,
        in_specs=[pl.BlockSpec((D, BC), lambda i: (0, i)) for _ in range(4)]
        + [pl.BlockSpec((L, D), lambda i: (0, 0))],
        out_specs=pl.BlockSpec((L, BC, DP), lambda i: (0, i, 0)),
        out_shape=jax.ShapeDtypeStruct((L, VEP, DP), jnp.float32),
    )(emb0.T, emb1.T, emb2.T, emb3.T, level_embed)
    return aug.reshape(L * VEP, DP)


def _idx_body(lv_ref, tk_ref, out_ref):
    out_ref[...] = lv_ref[...] * VEP + tk_ref[...]


def _build_idx(level_ids, token_ids):
    IB = 512
    gidx = pl.pallas_call(
        _idx_body,
        grid=(B // IB,),
        in_specs=[pl.BlockSpec((IB, T), lambda i: (i, 0)) for _ in range(2)],
        out_specs=pl.BlockSpec((IB, T), lambda i: (i, 0)),
        out_shape=jax.ShapeDtypeStruct((B, T), jnp.int32),
    )(level_ids, token_ids)
    return gidx.reshape(N // CH, CH)


_MESH = plsc.VectorSubcoreMesh(
    core_axis_name="c", subcore_axis_name="s", num_cores=NC, num_subcores=NS
)


@functools.partial(
    pl.kernel,
    out_type=jax.ShapeDtypeStruct((N, D), jnp.float32),
    mesh=_MESH,
    scratch_types=(
        [pltpu.VMEM((NCH, CH), jnp.int32)]
        + [pltpu.VMEM((CH, DP), jnp.float32) for _ in range(NBUF)]
        + [pltpu.SemaphoreType.DMA for _ in range(4)]
    ),
    compiler_params=pltpu.CompilerParams(use_tc_tiling_on_sc=False),
)
def _sc_gather(idx_hbm, table_hbm, out_hbm, idx_v,
               b0, b1, b2, b3, gs0, gs1, ws0, ws1):
    bufs = ((b0, b1), (b2, b3))
    gsem = (gs0, gs1)
    wsem = (ws0, ws1)
    wid = lax.axis_index("s") * NC + lax.axis_index("c")
    row0 = wid * NCH
    out0 = wid * PER_W

    pltpu.sync_copy(idx_hbm.at[pl.ds(row0, NCH)], idx_v)

    for g in range(2):
        for b in range(GRP):
            pltpu.async_copy(
                table_hbm.at[idx_v.at[g * GRP + b]], bufs[g][b], gsem[g]
            )

    def do_round(r, g):
        j0 = r * GRP
        for b in range(GRP):
            pltpu.make_async_copy(
                table_hbm.at[idx_v.at[0]], bufs[g][b], gsem[g]
            ).wait()
        for b in range(GRP):
            pltpu.async_copy(
                bufs[g][b].at[:, pl.ds(0, D)],
                out_hbm.at[pl.ds(out0 + (j0 + b) * CH, CH)],
                wsem[g],
            )
        for b in range(GRP):
            pltpu.make_async_copy(
                bufs[g][b].at[:, pl.ds(0, D)],
                out_hbm.at[pl.ds(out0, CH)],
                wsem[g],
            ).wait()
        nj0 = (r + 2) * GRP

        @pl.when(nj0 < NCH)
        def _():
            for b in range(GRP):
                pltpu.async_copy(
                    table_hbm.at[idx_v.at[nj0 + b]], bufs[g][b], gsem[g]
                )

    def outer(i, carry):
        do_round(2 * i, 0)
        do_round(2 * i + 1, 1)
        return carry

    lax.fori_loop(0, NROUND // 2, outer, None)


def kernel(level_ids, token_ids, emb0, emb1, emb2, emb3, level_embed):
    level_ids = level_ids.astype(jnp.int32)
    token_ids = token_ids.astype(jnp.int32)
    table = _build_table(emb0, emb1, emb2, emb3, level_embed)
    idx2d = _build_idx(level_ids, token_ids)
    out = _sc_gather(idx2d, table)
    return out.reshape(B, T, D)
